# baseline (speedup 1.0000x reference)
import functools

import jax
import jax.numpy as jnp
from jax import lax
from jax.experimental import pallas as pl
from jax.experimental.pallas import tpu as pltpu
from jax.experimental.pallas import tpu_sc as plsc

DIM = 64
B = 16384
LAMBD = 0.1
L = 16
CK = 16


def _tc_densify(in_ref, eye_ref, out_ref):
    x = in_ref[...]
    eye = eye_ref[...]
    dn = (((0,), (0,)), ((), ()))
    xe = jnp.concatenate([x[:, 512 * m:512 * m + 256] for m in range(8)],
                         axis=1)
    xo = jnp.concatenate([x[:, 512 * m + 256:512 * m + 512] for m in range(8)],
                         axis=1)
    out_ref[:, :DIM] = lax.dot_general(
        xe, eye, dn, preferred_element_type=jnp.float32)
    out_ref[:, DIM:] = lax.dot_general(
        xo, eye, dn, preferred_element_type=jnp.float32)


def _densify(table, cb=4096):
    n = table.shape[0]
    grid = ((n + cb - 1) // cb,)
    eye = jnp.eye(DIM, dtype=jnp.float32)
    return pl.pallas_call(
        _tc_densify,
        grid=grid,
        in_specs=[pl.BlockSpec((DIM, cb), lambda j: (0, j)),
                  pl.BlockSpec((DIM, DIM), lambda j: (0, 0))],
        out_specs=pl.BlockSpec((cb // 2, 128), lambda j: (j, 0)),
        out_shape=jax.ShapeDtypeStruct((n // 2, 128), jnp.float32),
    )(table.T, eye)


def _sc_make(nc, ns):
    nw = nc * ns
    rpw = B // nw
    nchunk = rpw // CK

    mesh = plsc.VectorSubcoreMesh(core_axis_name="c", subcore_axis_name="s")
    vec_f32 = jax.ShapeDtypeStruct((B,), jnp.float32)

    @functools.partial(
        pl.kernel,
        mesh=mesh,
        out_type=(vec_f32, vec_f32, vec_f32, vec_f32),
        compiler_params=pltpu.CompilerParams(
            needs_layout_passes=False, use_tc_tiling_on_sc=True),
        scratch_types=[
            pltpu.VMEM((rpw,), jnp.int32),
            pltpu.VMEM((rpw,), jnp.int32),
            pltpu.VMEM((2 * CK, 2 * DIM), jnp.float32),
            pltpu.VMEM((2 * CK, 2 * DIM), jnp.float32),
            pltpu.VMEM((2 * CK, 8, DIM), jnp.float32),
            pltpu.VMEM((rpw,), jnp.float32),
            pltpu.VMEM((rpw,), jnp.float32),
            pltpu.VMEM((rpw,), jnp.float32),
            pltpu.VMEM((rpw,), jnp.float32),
            pltpu.VMEM((L * L,), jnp.float32),
            pltpu.VMEM((L * L,), jnp.float32),
            pltpu.VMEM((L * L,), jnp.float32),
            pltpu.SemaphoreType.DMA,
            pltpu.SemaphoreType.DMA,
        ],
    )
    def sc_fn(src_hbm, ctx_hbm, psrc_hbm, pers_hbm,
              node_hbm, noise_hbm, base_hbm,
              ab_out, aa_out, bb_out, rd_out,
              idx_a, idx_b, a_pair, b_pair, b_grp,
              ab_v, aa_v, bb_v, rd_v, pab, paa, pbb, sem0, sem1):
        wid = lax.axis_index("s") * nc + lax.axis_index("c")
        base = pl.multiple_of(wid * rpw, rpw)
        col0 = lax.iota(jnp.int32, L) * L
        zero = jnp.zeros((L,), jnp.float32)

        def load_indices(ia_hbm, ib_hbm):
            pltpu.sync_copy(ia_hbm.at[pl.ds(base, rpw)], idx_a)
            pltpu.sync_copy(ib_hbm.at[pl.ds(base, rpw)], idx_b)

        def fetch_pair(t_hbm, buf, boff, k, i):
            pltpu.async_copy(t_hbm.at[pl.ds(i, 1)],
                             buf.at[pl.ds(boff + k, 1)],
                             sem0 if boff == 0 else sem1)

        def fetch_grp(t_hbm, buf, boff, k, i):
            pltpu.async_copy(t_hbm.at[pl.ds(i >> 3, 1)],
                             buf.at[pl.ds(boff + k, 1)],
                             sem0 if boff == 0 else sem1)

        def load_pair(buf, boff, r, i, q):
            return buf[boff + r, pl.ds(q * L, L)]

        def load_grp(buf, boff, r, i, q):
            return buf[boff + r, i & 7, pl.ds(q * L, L)]

        def phase(ta_hbm, tb_hbm, fetch_a, load_a, abuf, fetch_b, load_b,
                  bbuf, with_norms, out_main, out_aa, out_bb):
            def fire(t, boff):
                coff = t * CK
                iav = idx_a[pl.ds(coff, CK)]
                ibv = idx_b[pl.ds(coff, CK)]
                for k in range(CK):
                    fetch_a(ta_hbm, abuf, boff, k, iav[k])
                    fetch_b(tb_hbm, bbuf, boff, k, ibv[k])

            def drain(boff, sem):
                pltpu.make_async_copy(
                    ta_hbm.at[pl.ds(0, CK)],
                    abuf.at[pl.ds(boff, CK)], sem).wait()
                pltpu.make_async_copy(
                    tb_hbm.at[pl.ds(0, CK)],
                    bbuf.at[pl.ds(boff, CK)], sem).wait()

            def compute(t, boff):
                coff = t * CK
                iav = idx_a[pl.ds(coff, CK)]
                ibv = idx_b[pl.ds(coff, CK)]
                for r in range(L):
                    ia = iav[r]
                    ib = ibv[r]
                    ab_p, aa_p, bb_p = zero, zero, zero
                    for q in range(DIM // L):
                        a = load_a(abuf, boff, r, ia, q)
                        b = load_b(bbuf, boff, r, ib, q)
                        ab_p += a * b
                        if with_norms:
                            aa_p += a * a
                            bb_p += b * b
                    pab[pl.ds(r * L, L)] = ab_p
                    if with_norms:
                        paa[pl.ds(r * L, L)] = aa_p
                        pbb[pl.ds(r * L, L)] = bb_p
                ab, aa, bb = zero, zero, zero
                for c in range(L):
                    cidx = col0 + c
                    ab += plsc.load_gather(pab, [cidx])
                    if with_norms:
                        aa += plsc.load_gather(paa, [cidx])
                        bb += plsc.load_gather(pbb, [cidx])
                out_main[pl.ds(coff, L)] = ab
                if with_norms:
                    out_aa[pl.ds(coff, L)] = aa
                    out_bb[pl.ds(coff, L)] = bb

            fire(0, 0)

            def pair_body(tp, _):
                t0 = tp * 2
                t1 = t0 + 1
                fire(t1, CK)
                drain(0, sem0)
                compute(t0, 0)

                @pl.when(t1 + 1 < nchunk)
                def _():
                    fire(t1 + 1, 0)

                drain(CK, sem1)
                compute(t1, CK)
                return 0

            lax.fori_loop(0, nchunk // 2, pair_body, 0)

        load_indices(src_hbm, ctx_hbm)
        phase(node_hbm, noise_hbm, fetch_pair, load_pair, a_pair,
              fetch_grp, load_grp, b_grp, True, ab_v, aa_v, bb_v)

        load_indices(psrc_hbm, pers_hbm)
        phase(node_hbm, base_hbm, fetch_pair, load_pair, a_pair,
              fetch_pair, load_pair, b_pair, False, rd_v, None, None)

        obase = pl.multiple_of(base, 8)
        pltpu.sync_copy(ab_v, ab_out.at[pl.ds(obase, rpw)])
        pltpu.sync_copy(aa_v, aa_out.at[pl.ds(obase, rpw)])
        pltpu.sync_copy(bb_v, bb_out.at[pl.ds(obase, rpw)])
        pltpu.sync_copy(rd_v, rd_out.at[pl.ds(obase, rpw)])

    return sc_fn


def _tc_loss(t_ref, ab_ref, aa_ref, bb_ref, rd_ref, out_ref):
    ab = ab_ref[...]
    na = jnp.maximum(jnp.sqrt(aa_ref[...]), 1e-12)
    nb = jnp.maximum(jnp.sqrt(bb_ref[...]), 1e-12)
    s = jax.nn.sigmoid(ab / (na * nb))
    t = t_ref[...]
    main = t * jnp.log(s) + (1.0 - t) * jnp.log(1.0 - s)
    r = jnp.clip(rd_ref[...], -15.0, 15.0)
    rl = jnp.log(jax.nn.sigmoid(r))
    loss = -(jnp.sum(main) / B) - LAMBD * (jnp.sum(rl) / B)
    out_ref[...] = jnp.full((1, 1), loss, jnp.float32)


def kernel(sources, contexts, targets, personas, pure_sources,
           node_embedding, node_noise_embedding, base_node_embedding):
    info = plsc.get_sparse_core_info()
    sc_fn = _sc_make(info.num_cores, info.num_subcores)
    node2 = jnp.pad(node_embedding, ((0, 0), (0, DIM)))
    base2 = jnp.pad(base_node_embedding, ((0, 0), (0, DIM)))
    noise3 = node_noise_embedding.reshape(-1, 8, DIM)
    ab, aa, bb, rd = sc_fn(
        sources.astype(jnp.int32), contexts.astype(jnp.int32),
        pure_sources.astype(jnp.int32), personas.astype(jnp.int32),
        node2, noise3, base2)
    sh = (B // 128, 128)
    loss = pl.pallas_call(
        _tc_loss,
        out_shape=jax.ShapeDtypeStruct((1, 1), jnp.float32),
    )(targets.reshape(sh), ab.reshape(sh), aa.reshape(sh),
      bb.reshape(sh), rd.reshape(sh))
    return loss[0, 0]


# speedup vs baseline: 1.3791x; 1.3791x over previous
import functools

import jax
import jax.numpy as jnp
from jax import lax
from jax.experimental import pallas as pl
from jax.experimental.pallas import tpu as pltpu
from jax.experimental.pallas import tpu_sc as plsc

DIM = 64
B = 16384
LAMBD = 0.1
L = 16
CK = 16


def _sc_make(nc, ns):
    nw = nc * ns
    rpw = B // nw
    nchunk = rpw // CK

    mesh = plsc.VectorSubcoreMesh(core_axis_name="c", subcore_axis_name="s")
    vec_f32 = jax.ShapeDtypeStruct((B,), jnp.float32)

    @functools.partial(
        pl.kernel,
        mesh=mesh,
        out_type=(vec_f32, vec_f32, vec_f32, vec_f32),
        compiler_params=pltpu.CompilerParams(
            needs_layout_passes=False, use_tc_tiling_on_sc=True),
        scratch_types=[
            pltpu.VMEM((rpw,), jnp.int32),
            pltpu.VMEM((rpw,), jnp.int32),
            pltpu.VMEM((2 * CK, 8, DIM), jnp.float32),
            pltpu.VMEM((2 * CK, 8, DIM), jnp.float32),
            pltpu.VMEM((rpw,), jnp.float32),
            pltpu.VMEM((rpw,), jnp.float32),
            pltpu.VMEM((rpw,), jnp.float32),
            pltpu.VMEM((rpw,), jnp.float32),
            pltpu.VMEM((L * L,), jnp.float32),
            pltpu.VMEM((L * L,), jnp.float32),
            pltpu.VMEM((L * L,), jnp.float32),
            pltpu.SemaphoreType.DMA,
            pltpu.SemaphoreType.DMA,
        ],
    )
    def sc_fn(src_hbm, ctx_hbm, psrc_hbm, pers_hbm,
              node_hbm, noise_hbm, base_hbm,
              ab_out, aa_out, bb_out, rd_out,
              idx_a, idx_b, a_grp, b_grp,
              ab_v, aa_v, bb_v, rd_v, pab, paa, pbb, sem0, sem1):
        wid = lax.axis_index("s") * nc + lax.axis_index("c")
        base = pl.multiple_of(wid * rpw, rpw)
        col0 = lax.iota(jnp.int32, L) * L
        zero = jnp.zeros((L,), jnp.float32)

        def load_indices(ia_hbm, ib_hbm):
            pltpu.sync_copy(ia_hbm.at[pl.ds(base, rpw)], idx_a)
            pltpu.sync_copy(ib_hbm.at[pl.ds(base, rpw)], idx_b)

        def fetch_grp(t_hbm, buf, boff, k, i):
            pltpu.async_copy(t_hbm.at[pl.ds(i >> 3, 1)],
                             buf.at[pl.ds(boff + k, 1)],
                             sem0 if boff == 0 else sem1)

        def load_grp(buf, boff, r, i, q):
            return buf[boff + r, i & 7, pl.ds(q * L, L)]

        def phase(ta_hbm, tb_hbm, fetch_a, load_a, abuf, fetch_b, load_b,
                  bbuf, with_norms, out_main, out_aa, out_bb):
            def fire(t, boff):
                coff = t * CK
                iav = idx_a[pl.ds(coff, CK)]
                ibv = idx_b[pl.ds(coff, CK)]
                for k in range(CK):
                    fetch_a(ta_hbm, abuf, boff, k, iav[k])
                    fetch_b(tb_hbm, bbuf, boff, k, ibv[k])

            def drain(boff, sem):
                pltpu.make_async_copy(
                    ta_hbm.at[pl.ds(0, CK)],
                    abuf.at[pl.ds(boff, CK)], sem).wait()
                pltpu.make_async_copy(
                    tb_hbm.at[pl.ds(0, CK)],
                    bbuf.at[pl.ds(boff, CK)], sem).wait()

            def compute(t, boff):
                coff = t * CK
                iav = idx_a[pl.ds(coff, CK)]
                ibv = idx_b[pl.ds(coff, CK)]
                for r in range(L):
                    ia = iav[r]
                    ib = ibv[r]
                    ab_p, aa_p, bb_p = zero, zero, zero
                    for q in range(DIM // L):
                        a = load_a(abuf, boff, r, ia, q)
                        b = load_b(bbuf, boff, r, ib, q)
                        ab_p += a * b
                        if with_norms:
                            aa_p += a * a
                            bb_p += b * b
                    pab[pl.ds(r * L, L)] = ab_p
                    if with_norms:
                        paa[pl.ds(r * L, L)] = aa_p
                        pbb[pl.ds(r * L, L)] = bb_p
                ab, aa, bb = zero, zero, zero
                for c in range(L):
                    cidx = col0 + c
                    ab += plsc.load_gather(pab, [cidx])
                    if with_norms:
                        aa += plsc.load_gather(paa, [cidx])
                        bb += plsc.load_gather(pbb, [cidx])
                out_main[pl.ds(coff, L)] = ab
                if with_norms:
                    out_aa[pl.ds(coff, L)] = aa
                    out_bb[pl.ds(coff, L)] = bb

            fire(0, 0)

            def pair_body(tp, _):
                t0 = tp * 2
                t1 = t0 + 1
                fire(t1, CK)
                drain(0, sem0)
                compute(t0, 0)

                @pl.when(t1 + 1 < nchunk)
                def _():
                    fire(t1 + 1, 0)

                drain(CK, sem1)
                compute(t1, CK)
                return 0

            lax.fori_loop(0, nchunk // 2, pair_body, 0)

        load_indices(src_hbm, ctx_hbm)
        phase(node_hbm, noise_hbm, fetch_grp, load_grp, a_grp,
              fetch_grp, load_grp, b_grp, True, ab_v, aa_v, bb_v)

        load_indices(psrc_hbm, pers_hbm)
        phase(node_hbm, base_hbm, fetch_grp, load_grp, a_grp,
              fetch_grp, load_grp, b_grp, False, rd_v, None, None)

        obase = pl.multiple_of(base, 8)
        pltpu.sync_copy(ab_v, ab_out.at[pl.ds(obase, rpw)])
        pltpu.sync_copy(aa_v, aa_out.at[pl.ds(obase, rpw)])
        pltpu.sync_copy(bb_v, bb_out.at[pl.ds(obase, rpw)])
        pltpu.sync_copy(rd_v, rd_out.at[pl.ds(obase, rpw)])

    return sc_fn


def _tc_loss(t_ref, ab_ref, aa_ref, bb_ref, rd_ref, out_ref):
    ab = ab_ref[...]
    na = jnp.maximum(jnp.sqrt(aa_ref[...]), 1e-12)
    nb = jnp.maximum(jnp.sqrt(bb_ref[...]), 1e-12)
    s = jax.nn.sigmoid(ab / (na * nb))
    t = t_ref[...]
    main = t * jnp.log(s) + (1.0 - t) * jnp.log(1.0 - s)
    r = jnp.clip(rd_ref[...], -15.0, 15.0)
    rl = jnp.log(jax.nn.sigmoid(r))
    loss = -(jnp.sum(main) / B) - LAMBD * (jnp.sum(rl) / B)
    out_ref[...] = jnp.full((1, 1), loss, jnp.float32)


def kernel(sources, contexts, targets, personas, pure_sources,
           node_embedding, node_noise_embedding, base_node_embedding):
    info = plsc.get_sparse_core_info()
    sc_fn = _sc_make(info.num_cores, info.num_subcores)
    node3 = node_embedding.reshape(-1, 8, DIM)
    noise3 = node_noise_embedding.reshape(-1, 8, DIM)
    base3 = base_node_embedding.reshape(-1, 8, DIM)
    ab, aa, bb, rd = sc_fn(
        sources.astype(jnp.int32), contexts.astype(jnp.int32),
        pure_sources.astype(jnp.int32), personas.astype(jnp.int32),
        node3, noise3, base3)
    sh = (B // 128, 128)
    loss = pl.pallas_call(
        _tc_loss,
        out_shape=jax.ShapeDtypeStruct((1, 1), jnp.float32),
    )(targets.reshape(sh), ab.reshape(sh), aa.reshape(sh),
      bb.reshape(sh), rd.reshape(sh))
    return loss[0, 0]


# speedup vs baseline: 1.4025x; 1.0170x over previous
import functools

import jax
import jax.numpy as jnp
from jax import lax
from jax.experimental import pallas as pl
from jax.experimental.pallas import tpu as pltpu
from jax.experimental.pallas import tpu_sc as plsc

DIM = 64
B = 16384
LAMBD = 0.1
L = 16
CK = 16


def _sc_make(nc, ns):
    nw = nc * ns
    rpw = B // nw
    nchunk = rpw // CK

    mesh = plsc.VectorSubcoreMesh(core_axis_name="c", subcore_axis_name="s")
    vec_f32 = jax.ShapeDtypeStruct((B,), jnp.float32)

    @functools.partial(
        pl.kernel,
        mesh=mesh,
        out_type=(vec_f32, vec_f32, vec_f32, vec_f32),
        compiler_params=pltpu.CompilerParams(
            needs_layout_passes=False, use_tc_tiling_on_sc=True),
        scratch_types=[
            pltpu.VMEM((rpw,), jnp.int32),
            pltpu.VMEM((rpw,), jnp.int32),
            pltpu.VMEM((3 * CK, 8, DIM), jnp.float32),
            pltpu.VMEM((3 * CK, 8, DIM), jnp.float32),
            pltpu.VMEM((rpw,), jnp.float32),
            pltpu.VMEM((rpw,), jnp.float32),
            pltpu.VMEM((rpw,), jnp.float32),
            pltpu.VMEM((rpw,), jnp.float32),
            pltpu.VMEM((L * L,), jnp.float32),
            pltpu.VMEM((L * L,), jnp.float32),
            pltpu.VMEM((L * L,), jnp.float32),
            pltpu.SemaphoreType.DMA,
            pltpu.SemaphoreType.DMA,
            pltpu.SemaphoreType.DMA,
        ],
    )
    def sc_fn(src_hbm, ctx_hbm, psrc_hbm, pers_hbm,
              node_hbm, noise_hbm, base_hbm,
              ab_out, aa_out, bb_out, rd_out,
              idx_a, idx_b, a_grp, b_grp,
              ab_v, aa_v, bb_v, rd_v, pab, paa, pbb, sem0, sem1, sem2):
        wid = lax.axis_index("s") * nc + lax.axis_index("c")
        base = pl.multiple_of(wid * rpw, rpw)
        col0 = lax.iota(jnp.int32, L) * L
        zero = jnp.zeros((L,), jnp.float32)

        def load_indices(ia_hbm, ib_hbm):
            pltpu.sync_copy(ia_hbm.at[pl.ds(base, rpw)], idx_a)
            pltpu.sync_copy(ib_hbm.at[pl.ds(base, rpw)], idx_b)

        sems = (sem0, sem1, sem2)

        def fetch_grp(t_hbm, buf, p, k, i):
            pltpu.async_copy(t_hbm.at[pl.ds(i >> 3, 1)],
                             buf.at[pl.ds(p * CK + k, 1)],
                             sems[p])

        def load_grp(buf, boff, r, i, q):
            return buf[boff + r, i & 7, pl.ds(q * L, L)]

        def phase(ta_hbm, tb_hbm, fetch_a, load_a, abuf, fetch_b, load_b,
                  bbuf, with_norms, out_main, out_aa, out_bb):
            def fire(t, p):
                coff = t * CK
                iav = idx_a[pl.ds(coff, CK)]
                ibv = idx_b[pl.ds(coff, CK)]
                for k in range(CK):
                    fetch_a(ta_hbm, abuf, p, k, iav[k])
                    fetch_b(tb_hbm, bbuf, p, k, ibv[k])

            def drain(p):
                pltpu.make_async_copy(
                    ta_hbm.at[pl.ds(0, CK)],
                    abuf.at[pl.ds(p * CK, CK)], sems[p]).wait()
                pltpu.make_async_copy(
                    tb_hbm.at[pl.ds(0, CK)],
                    bbuf.at[pl.ds(p * CK, CK)], sems[p]).wait()

            def compute(t, boff):
                coff = t * CK
                iav = idx_a[pl.ds(coff, CK)]
                ibv = idx_b[pl.ds(coff, CK)]
                for r in range(L):
                    ia = iav[r]
                    ib = ibv[r]
                    ab_p, aa_p, bb_p = zero, zero, zero
                    for q in range(DIM // L):
                        a = load_a(abuf, boff, r, ia, q)
                        b = load_b(bbuf, boff, r, ib, q)
                        ab_p += a * b
                        if with_norms:
                            aa_p += a * a
                            bb_p += b * b
                    pab[pl.ds(r * L, L)] = ab_p
                    if with_norms:
                        paa[pl.ds(r * L, L)] = aa_p
                        pbb[pl.ds(r * L, L)] = bb_p
                ab, aa, bb = zero, zero, zero
                for c in range(L):
                    cidx = col0 + c
                    ab += plsc.load_gather(pab, [cidx])
                    if with_norms:
                        aa += plsc.load_gather(paa, [cidx])
                        bb += plsc.load_gather(pbb, [cidx])
                out_main[pl.ds(coff, L)] = ab
                if with_norms:
                    out_aa[pl.ds(coff, L)] = aa
                    out_bb[pl.ds(coff, L)] = bb

            fire(0, 0)
            fire(1, 1)

            def tri_body(tg, _):
                for j in range(3):
                    t = tg * 3 + j

                    @pl.when(t + 2 < nchunk)
                    def _():
                        fire(t + 2, (j + 2) % 3)

                    @pl.when(t < nchunk)
                    def _():
                        drain(j)
                        compute(t, j * CK)

                return 0

            lax.fori_loop(0, (nchunk + 2) // 3, tri_body, 0)

        load_indices(src_hbm, ctx_hbm)
        phase(node_hbm, noise_hbm, fetch_grp, load_grp, a_grp,
              fetch_grp, load_grp, b_grp, True, ab_v, aa_v, bb_v)

        load_indices(psrc_hbm, pers_hbm)
        phase(node_hbm, base_hbm, fetch_grp, load_grp, a_grp,
              fetch_grp, load_grp, b_grp, False, rd_v, None, None)

        obase = pl.multiple_of(base, 8)
        pltpu.sync_copy(ab_v, ab_out.at[pl.ds(obase, rpw)])
        pltpu.sync_copy(aa_v, aa_out.at[pl.ds(obase, rpw)])
        pltpu.sync_copy(bb_v, bb_out.at[pl.ds(obase, rpw)])
        pltpu.sync_copy(rd_v, rd_out.at[pl.ds(obase, rpw)])

    return sc_fn


def _tc_loss(t_ref, ab_ref, aa_ref, bb_ref, rd_ref, out_ref):
    ab = ab_ref[...]
    na = jnp.maximum(jnp.sqrt(aa_ref[...]), 1e-12)
    nb = jnp.maximum(jnp.sqrt(bb_ref[...]), 1e-12)
    s = jax.nn.sigmoid(ab / (na * nb))
    t = t_ref[...]
    main = t * jnp.log(s) + (1.0 - t) * jnp.log(1.0 - s)
    r = jnp.clip(rd_ref[...], -15.0, 15.0)
    rl = jnp.log(jax.nn.sigmoid(r))
    loss = -(jnp.sum(main) / B) - LAMBD * (jnp.sum(rl) / B)
    out_ref[...] = jnp.full((1, 1), loss, jnp.float32)


def kernel(sources, contexts, targets, personas, pure_sources,
           node_embedding, node_noise_embedding, base_node_embedding):
    info = plsc.get_sparse_core_info()
    sc_fn = _sc_make(info.num_cores, info.num_subcores)
    node3 = node_embedding.reshape(-1, 8, DIM)
    noise3 = node_noise_embedding.reshape(-1, 8, DIM)
    base3 = base_node_embedding.reshape(-1, 8, DIM)
    ab, aa, bb, rd = sc_fn(
        sources.astype(jnp.int32), contexts.astype(jnp.int32),
        pure_sources.astype(jnp.int32), personas.astype(jnp.int32),
        node3, noise3, base3)
    sh = (B // 128, 128)
    loss = pl.pallas_call(
        _tc_loss,
        out_shape=jax.ShapeDtypeStruct((1, 1), jnp.float32),
    )(targets.reshape(sh), ab.reshape(sh), aa.reshape(sh),
      bb.reshape(sh), rd.reshape(sh))
    return loss[0, 0]
